# baseline (speedup 1.0000x reference)
import functools

import jax
import jax.numpy as jnp
from jax import lax
from jax.experimental import pallas as pl
from jax.experimental.pallas import tpu as pltpu
from jax.experimental.pallas import tpu_sc as plsc

N = 512
E = 8192
H = 16
D = 256
NEG = -200000000.0

_MESH = plsc.VectorSubcoreMesh(core_axis_name="c", subcore_axis_name="s")


def _adj_body(ei_hbm, adj_hbm, ei_v, slab_v, sem):
    wid = lax.axis_index("s") * 2 + lax.axis_index("c")
    lo = wid * 16
    ld = pltpu.async_copy(ei_hbm, ei_v, sem)
    zeros = jnp.zeros((16,), jnp.int32)

    @plsc.parallel_loop(0, 512, unroll=4)
    def zb(c):
        slab_v[pl.ds(c * 16, 16)] = zeros

    ld.wait()
    ones = jnp.ones((16,), jnp.int32)

    @plsc.parallel_loop(0, E // 16, unroll=4)
    def ebody(e):
        s = ei_v[0, pl.ds(e * 16, 16)]
        d = ei_v[1, pl.ds(e * 16, 16)]
        m = (s >= lo) & (s < lo + 16)
        plsc.store_scatter(slab_v, [(s - lo) * N + d], ones, mask=m)

    for q in range(4):
        pltpu.sync_copy(slab_v.at[pl.ds(q * 128, 128)],
                        adj_hbm.at[q, lo, :])
        for r in range(1, 16):
            pltpu.sync_copy(slab_v.at[pl.ds(r * N + q * 128, 128)],
                            adj_hbm.at[q, lo + r, :])


_adj_call = functools.partial(
    pl.kernel,
    out_type=jax.ShapeDtypeStruct((4, N, 128), jnp.int32),
    mesh=_MESH,
    compiler_params=pltpu.CompilerParams(use_tc_tiling_on_sc=False,
                                         needs_layout_passes=False),
    scratch_types=[
        pltpu.VMEM((2, E), jnp.int32),
        pltpu.VMEM((16 * N,), jnp.int32),
        pltpu.SemaphoreType.DMA,
    ],
)(_adj_body)


def _tc_body(adj_ref, x_ref, it_ref, ot_ref, rpt_ref, vd_ref,
             relp_ref, nf_ref, augt_ref, reach_ref,
             adjb_ref, rel_ref):
    adj = jnp.concatenate([adj_ref[q] for q in range(4)], axis=1)
    adjb_ref[...] = adj.astype(jnp.bfloat16)
    row = lax.broadcasted_iota(jnp.int32, (N, N), 0)
    col = lax.broadcasted_iota(jnp.int32, (N, N), 1)
    reach_ref[...] = (row == col).astype(jnp.float32)
    rel_ref[...] = jnp.zeros((N, N), jnp.int32)

    def cond(c):
        return c[1] & (c[0] < N)

    def body(c):
        t, _ = c
        reach = reach_ref[...]
        rel_ref[...] = rel_ref[...] + (reach == 0).astype(jnp.int32)
        prod = jnp.dot(reach.astype(jnp.bfloat16), adjb_ref[...],
                       preferred_element_type=jnp.float32)
        new = jnp.where(prod + reach > 0.0, jnp.float32(1), jnp.float32(0))
        reach_ref[...] = new
        return (t + 1, jnp.any(new != reach))

    lax.while_loop(cond, body, (jnp.int32(0), jnp.bool_(True)))

    relc = jnp.where(reach_ref[...] > 0,
                     jnp.minimum(rel_ref[...], 20), 20)
    relp_ref[pl.ds(1, N), :] = relc
    relp_ref[pl.ds(0, 1), :] = jnp.full((1, N), 21, jnp.int32)
    relp_ref[pl.ds(N + 1, 127), :] = jnp.full((127, N), 21, jnp.int32)

    onesc = jnp.ones((N, 1), jnp.bfloat16)
    adjb = adjb_ref[...]
    in_deg = lax.dot_general(adjb, onesc, (((1,), (0,)), ((), ())),
                             preferred_element_type=jnp.float32)
    out_deg = lax.dot_general(adjb, onesc, (((0,), (0,)), ((), ())),
                              preferred_element_type=jnp.float32)
    in_deg = jnp.minimum(in_deg, 511.0).astype(jnp.int32)
    out_deg = jnp.minimum(out_deg, 511.0).astype(jnp.int32)
    oh_in = (in_deg == col).astype(jnp.bfloat16)
    oh_out = (out_deg == col).astype(jnp.bfloat16)
    it = it_ref[...]
    ot = ot_ref[...]
    it_hi = it.astype(jnp.bfloat16)
    ot_hi = ot.astype(jnp.bfloat16)
    it_lo = (it - it_hi.astype(jnp.float32)).astype(jnp.bfloat16)
    ot_lo = (ot - ot_hi.astype(jnp.float32)).astype(jnp.bfloat16)
    nf_ref[...] = (x_ref[...]
                   + jnp.dot(oh_in, it_hi,
                             preferred_element_type=jnp.float32)
                   + jnp.dot(oh_in, it_lo,
                             preferred_element_type=jnp.float32)
                   + jnp.dot(oh_out, ot_hi,
                             preferred_element_type=jnp.float32)
                   + jnp.dot(oh_out, ot_lo,
                             preferred_element_type=jnp.float32))

    t32 = jnp.transpose(rpt_ref[0:24, :])
    vdc = jnp.transpose(vd_ref[...])
    vl = lax.broadcasted_iota(jnp.int32, (H, 24), 1)
    augt = jnp.where(vl >= 21, jnp.float32(0.0), t32 + vdc)
    hi = augt.astype(jnp.bfloat16)
    lo = (augt - hi.astype(jnp.float32)).astype(jnp.bfloat16)
    augt_ref[...] = jnp.concatenate([hi, lo], axis=0)


def _tc_call(adj, x, it, ot, rpt, vd):
    return pl.pallas_call(
        _tc_body,
        out_shape=(
            jax.ShapeDtypeStruct((N + 128, N), jnp.int32),
            jax.ShapeDtypeStruct((N, D), jnp.float32),
            jax.ShapeDtypeStruct((2 * H, 24), jnp.bfloat16),
        ),
        scratch_shapes=[
            pltpu.VMEM((N, N), jnp.float32),
            pltpu.VMEM((N, N), jnp.bfloat16),
            pltpu.VMEM((N, N), jnp.int32),
        ],
    )(adj, x, it, ot, rpt, vd)


IB = 128


def _gab_tc_body(relp_ref, augt_ref, out_ref):
    relb = relp_ref[...]
    iota_v = lax.broadcasted_iota(jnp.int32, (24, N), 0)
    parts = []
    negs = []
    for r in range(IB):
        rowr = relb[r:r + 1, :]
        ohr = (jnp.broadcast_to(rowr, (24, N)) == iota_v)
        parts.append(jnp.where(ohr, jnp.float32(1.0), jnp.float32(0.0))
                     .astype(jnp.bfloat16))
        negs.append(jnp.where(jnp.broadcast_to(rowr == 20, (H, N)),
                              jnp.float32(NEG), jnp.float32(0.0)))
    oh = jnp.concatenate(parts, axis=1)
    res2 = jnp.dot(augt_ref[...], oh,
                   preferred_element_type=jnp.float32)
    res = res2[0:H, :] + res2[H:2 * H, :]
    for r in range(IB):
        out_ref[pl.ds(r * H, H), pl.ds(1, N)] = (
            res[:, r * N:(r + 1) * N] + negs[r])
    out_ref[:, pl.ds(0, 1)] = jnp.zeros((IB * H, 1), jnp.float32)


def _gab_tc_call(relp, augt):
    nsteps = ((N + 1) * H + IB * H - 1) // (IB * H)
    return pl.pallas_call(
        _gab_tc_body,
        grid=(nsteps,),
        in_specs=[
            pl.BlockSpec((IB, N), lambda g: (g, 0)),
            pl.BlockSpec((2 * H, 24), lambda g: (0, 0)),
        ],
        out_specs=pl.BlockSpec((IB * H, N + 1), lambda g: (g, 0)),
        out_shape=jax.ShapeDtypeStruct(((N + 1) * H, N + 1), jnp.float32),
    )(relp, augt)


def kernel(x, edge_feature, edge_index, in_deg_table, out_deg_table,
           rel_pos_table, virt_dist):
    del edge_feature
    adj = _adj_call(edge_index.astype(jnp.int32))
    relp, node_feature, augt = _tc_call(
        adj, x, in_deg_table, out_deg_table, rel_pos_table, virt_dist)
    gab2d = _gab_tc_call(relp, augt)
    gab = gab2d.reshape(N + 1, H, N + 1)
    return node_feature, gab


# speedup vs baseline: 1.0557x; 1.0557x over previous
import functools

import jax
import jax.numpy as jnp
from jax import lax
from jax.experimental import pallas as pl
from jax.experimental.pallas import tpu as pltpu
from jax.experimental.pallas import tpu_sc as plsc

N = 512
E = 8192
H = 16
D = 256
NEG = -200000000.0

_MESH = plsc.VectorSubcoreMesh(core_axis_name="c", subcore_axis_name="s")


def _adj_body(ei_hbm, adj_hbm, ei_v, slab_v, sem):
    wid = lax.axis_index("s") * 2 + lax.axis_index("c")
    lo = wid * 16
    ld = pltpu.async_copy(ei_hbm, ei_v, sem)
    zeros = jnp.zeros((16,), jnp.int32)
    for q in range(4):
        for r in range(16):
            @plsc.parallel_loop(0, 8, unroll=4)
            def zb(c, q=q, r=r):
                slab_v[q, r, pl.ds(c * 16, 16)] = zeros

    ld.wait()
    ones = jnp.ones((16,), jnp.int32)

    @plsc.parallel_loop(0, E // 16, unroll=4)
    def ebody(e):
        s = ei_v[0, pl.ds(e * 16, 16)]
        d = ei_v[1, pl.ds(e * 16, 16)]
        m = (s >= lo) & (s < lo + 16)
        plsc.store_scatter(slab_v, [d >> 7, s - lo, d & 127], ones,
                           mask=m)

    for q in range(4):
        pltpu.sync_copy(slab_v.at[q], adj_hbm.at[q, pl.ds(lo, 16), :])


_adj_call = functools.partial(
    pl.kernel,
    out_type=jax.ShapeDtypeStruct((4, N, 128), jnp.int32),
    mesh=_MESH,
    compiler_params=pltpu.CompilerParams(use_tc_tiling_on_sc=False,
                                         needs_layout_passes=False),
    scratch_types=[
        pltpu.VMEM((2, E), jnp.int32),
        pltpu.VMEM((4, 16, 128), jnp.int32),
        pltpu.SemaphoreType.DMA,
    ],
)(_adj_body)


def _tc_body(adj_ref, x_ref, it_ref, ot_ref, rpt_ref, vd_ref,
             relp_ref, nf_ref, augt_ref, reach_ref,
             adjb_ref, rel_ref):
    adj = jnp.concatenate([adj_ref[q] for q in range(4)], axis=1)
    adjb_ref[...] = adj.astype(jnp.bfloat16)
    row = lax.broadcasted_iota(jnp.int32, (N, N), 0)
    col = lax.broadcasted_iota(jnp.int32, (N, N), 1)
    reach_ref[...] = (row == col).astype(jnp.float32)
    rel_ref[...] = jnp.zeros((N, N), jnp.int32)

    def cond(c):
        return c[1] & (c[0] < N)

    def body(c):
        t, _ = c
        reach = reach_ref[...]
        rel_ref[...] = rel_ref[...] + (reach == 0).astype(jnp.int32)
        prod = jnp.dot(reach.astype(jnp.bfloat16), adjb_ref[...],
                       preferred_element_type=jnp.float32)
        new = jnp.where(prod + reach > 0.0, jnp.float32(1), jnp.float32(0))
        reach_ref[...] = new
        return (t + 1, jnp.any(new != reach))

    lax.while_loop(cond, body, (jnp.int32(0), jnp.bool_(True)))

    relc = jnp.where(reach_ref[...] > 0,
                     jnp.minimum(rel_ref[...], 20), 20)
    relp_ref[pl.ds(1, N), :] = relc
    relp_ref[pl.ds(0, 1), :] = jnp.full((1, N), 21, jnp.int32)
    relp_ref[pl.ds(N + 1, 127), :] = jnp.full((127, N), 21, jnp.int32)

    onesc = jnp.ones((N, 1), jnp.bfloat16)
    adjb = adjb_ref[...]
    in_deg = lax.dot_general(adjb, onesc, (((1,), (0,)), ((), ())),
                             preferred_element_type=jnp.float32)
    out_deg = lax.dot_general(adjb, onesc, (((0,), (0,)), ((), ())),
                              preferred_element_type=jnp.float32)
    in_deg = jnp.minimum(in_deg, 511.0).astype(jnp.int32)
    out_deg = jnp.minimum(out_deg, 511.0).astype(jnp.int32)
    oh_in = (in_deg == col).astype(jnp.bfloat16)
    oh_out = (out_deg == col).astype(jnp.bfloat16)
    it = it_ref[...]
    ot = ot_ref[...]
    it_hi = it.astype(jnp.bfloat16)
    ot_hi = ot.astype(jnp.bfloat16)
    it_lo = (it - it_hi.astype(jnp.float32)).astype(jnp.bfloat16)
    ot_lo = (ot - ot_hi.astype(jnp.float32)).astype(jnp.bfloat16)
    nf_ref[...] = (x_ref[...]
                   + jnp.dot(oh_in, it_hi,
                             preferred_element_type=jnp.float32)
                   + jnp.dot(oh_in, it_lo,
                             preferred_element_type=jnp.float32)
                   + jnp.dot(oh_out, ot_hi,
                             preferred_element_type=jnp.float32)
                   + jnp.dot(oh_out, ot_lo,
                             preferred_element_type=jnp.float32))

    t32 = jnp.transpose(rpt_ref[0:24, :])
    vdc = jnp.transpose(vd_ref[...])
    vl = lax.broadcasted_iota(jnp.int32, (H, 24), 1)
    augt = jnp.where(vl >= 21, jnp.float32(0.0), t32 + vdc)
    hi = augt.astype(jnp.bfloat16)
    lo = (augt - hi.astype(jnp.float32)).astype(jnp.bfloat16)
    augt_ref[...] = jnp.concatenate([hi, lo], axis=0)


def _tc_call(adj, x, it, ot, rpt, vd):
    return pl.pallas_call(
        _tc_body,
        out_shape=(
            jax.ShapeDtypeStruct((N + 128, N), jnp.int32),
            jax.ShapeDtypeStruct((N, D), jnp.float32),
            jax.ShapeDtypeStruct((2 * H, 24), jnp.bfloat16),
        ),
        scratch_shapes=[
            pltpu.VMEM((N, N), jnp.float32),
            pltpu.VMEM((N, N), jnp.bfloat16),
            pltpu.VMEM((N, N), jnp.int32),
        ],
    )(adj, x, it, ot, rpt, vd)


IB = 128


def _gab_tc_body(relp_ref, augt_ref, out_ref):
    relb = relp_ref[...]
    iota_v = lax.broadcasted_iota(jnp.int32, (24, N), 0)
    parts = []
    negs = []
    for r in range(IB):
        rowr = relb[r:r + 1, :]
        ohr = (jnp.broadcast_to(rowr, (24, N)) == iota_v)
        parts.append(jnp.where(ohr, jnp.float32(1.0), jnp.float32(0.0))
                     .astype(jnp.bfloat16))
        negs.append(jnp.where(jnp.broadcast_to(rowr == 20, (H, N)),
                              jnp.float32(NEG), jnp.float32(0.0)))
    oh = jnp.concatenate(parts, axis=1)
    res2 = jnp.dot(augt_ref[...], oh,
                   preferred_element_type=jnp.float32)
    res = res2[0:H, :] + res2[H:2 * H, :]
    for r in range(IB):
        out_ref[pl.ds(r * H, H), pl.ds(1, N)] = (
            res[:, r * N:(r + 1) * N] + negs[r])
    out_ref[:, pl.ds(0, 1)] = jnp.zeros((IB * H, 1), jnp.float32)


def _gab_tc_call(relp, augt):
    nsteps = ((N + 1) * H + IB * H - 1) // (IB * H)
    return pl.pallas_call(
        _gab_tc_body,
        grid=(nsteps,),
        in_specs=[
            pl.BlockSpec((IB, N), lambda g: (g, 0)),
            pl.BlockSpec((2 * H, 24), lambda g: (0, 0)),
        ],
        out_specs=pl.BlockSpec((IB * H, N + 1), lambda g: (g, 0)),
        out_shape=jax.ShapeDtypeStruct(((N + 1) * H, N + 1), jnp.float32),
    )(relp, augt)


def kernel(x, edge_feature, edge_index, in_deg_table, out_deg_table,
           rel_pos_table, virt_dist):
    del edge_feature
    adj = _adj_call(edge_index.astype(jnp.int32))
    relp, node_feature, augt = _tc_call(
        adj, x, in_deg_table, out_deg_table, rel_pos_table, virt_dist)
    gab2d = _gab_tc_call(relp, augt)
    gab = gab2d.reshape(N + 1, H, N + 1)
    return node_feature, gab
